# Initial kernel scaffold; baseline (speedup 1.0000x reference)
import jax
import jax.numpy as jnp
from jax.experimental import pallas as pl


def kernel(x, gate_w, eg, eu, ed, sg, su, sd):
    raise NotImplementedError("write your pallas kernel here")


# speedup vs baseline: 7.0147x; 7.0147x over previous
import functools

import jax
import jax.numpy as jnp
from jax import lax
from jax.experimental import pallas as pl
from jax.experimental.pallas import tpu as pltpu
from jax.experimental.pallas import tpu_sc as plsc

D = 1024
DFF = 512
E = 64
TOPK = 2

NC = 2
NS = 16
NW = NC * NS

RTM = 512
TM = 256

_f32 = jnp.float32
_i32 = jnp.int32


def _silu(g):
    return g * (1.0 / (1.0 + jnp.exp(-g)))


def _dot_nt(a, b, precision=None):
    return lax.dot_general(a, b, (((1,), (1,)), ((), ())),
                           preferred_element_type=_f32, precision=precision)


def _router_shared_body(x_ref, gw_ref, sg_ref, su_ref, sd_ref,
                        ysh_ref, e0_ref, e1_ref, r0_ref, r1_ref,
                        w0_ref, w1_ref, cnt_ref, off_ref, carry_ref):
    i = pl.program_id(0)
    nsteps = pl.num_programs(0)
    xb = x_ref[...]

    g = _dot_nt(xb, sg_ref[...])
    u = _dot_nt(xb, su_ref[...])
    ysh_ref[...] = _dot_nt(_silu(g) * u, sd_ref[...])

    logits = _dot_nt(xb.astype(jnp.bfloat16),
                     gw_ref[...].astype(jnp.bfloat16))
    m = jnp.max(logits, axis=1, keepdims=True)
    p = jnp.exp(logits - m)
    p = p / jnp.sum(p, axis=1, keepdims=True)
    iot = lax.broadcasted_iota(_i32, (RTM, E), 1)
    p0 = jnp.max(p, axis=1, keepdims=True)
    i0 = jnp.min(jnp.where(p == p0, iot, E), axis=1, keepdims=True)
    pm = jnp.where(iot == i0, -1.0, p)
    p1 = jnp.max(pm, axis=1, keepdims=True)
    i1 = jnp.min(jnp.where(pm == p1, iot, E), axis=1, keepdims=True)
    s = p0 + p1 + 1e-6
    w0 = p0 / s
    w1 = p1 / s

    oh = (iot == i0).astype(_f32) + (iot == i1).astype(_f32)
    ri = lax.broadcasted_iota(_i32, (RTM, RTM), 0)
    ci = lax.broadcasted_iota(_i32, (RTM, RTM), 1)
    ltri = (ri > ci).astype(_f32)
    excl = lax.dot_general(ltri, oh, (((1,), (0,)), ((), ())),
                           preferred_element_type=_f32,
                           precision=lax.Precision.HIGHEST)

    @pl.when(i == 0)
    def _():
        carry_ref[...] = jnp.zeros_like(carry_ref)

    carry = carry_ref[0:1, 0:E]
    tot = excl + carry
    r0 = jnp.sum(jnp.where(iot == i0, tot, 0.0), axis=1, keepdims=True)
    r1 = jnp.sum(jnp.where(iot == i1, tot, 0.0), axis=1, keepdims=True)
    newc = carry + jnp.sum(oh, axis=0, keepdims=True)
    carry_ref[0:1, 0:E] = newc

    e0_ref[...] = i0
    e1_ref[...] = i1
    r0_ref[...] = r0.astype(_i32)
    r1_ref[...] = r1.astype(_i32)
    w0_ref[...] = w0
    w1_ref[...] = w1

    @pl.when(i == nsteps - 1)
    def _():
        er = lax.broadcasted_iota(_i32, (E, E), 0)
        ec = lax.broadcasted_iota(_i32, (E, E), 1)
        le = (ec < er).astype(_f32)
        offs = lax.dot_general(newc, le, (((1,), (1,)), ((), ())),
                               preferred_element_type=_f32,
                               precision=lax.Precision.HIGHEST)
        cnt_ref[...] = newc.astype(_i32)
        off_ref[...] = offs.astype(_i32)


def _router_shared(x_flat, gate_w, sg2, su2, sd2):
    n = x_flat.shape[0]
    grid = (n // RTM,)
    col = lambda shape: jax.ShapeDtypeStruct(shape, _i32)
    out_shapes = (
        jax.ShapeDtypeStruct((n, D), _f32),
        col((n, 1)), col((n, 1)),
        col((n, 1)), col((n, 1)),
        jax.ShapeDtypeStruct((n, 1), _f32),
        jax.ShapeDtypeStruct((n, 1), _f32),
        col((1, E)), col((1, E)),
    )
    tok_spec = pl.BlockSpec((RTM, 1), lambda i: (i, 0))
    full = lambda shape: pl.BlockSpec(shape, lambda i: tuple(0 for _ in shape))
    return pl.pallas_call(
        _router_shared_body,
        grid=grid,
        in_specs=[
            pl.BlockSpec((RTM, D), lambda i: (i, 0)),
            full((E, D)),
            full((DFF, D)), full((DFF, D)), full((D, DFF)),
        ],
        out_specs=(
            pl.BlockSpec((RTM, D), lambda i: (i, 0)),
            tok_spec, tok_spec, tok_spec, tok_spec, tok_spec, tok_spec,
            full((1, E)), full((1, E)),
        ),
        out_shape=out_shapes,
        scratch_shapes=[pltpu.VMEM((8, 128), _f32)],
    )(x_flat, gate_w, sg2, su2, sd2)


def _build_meta(counts, offsets, t_tiles, gmax):
    counts = counts.reshape(-1).astype(_i32)
    offsets = offsets.reshape(-1).astype(_i32)
    ts = offsets // TM
    te = jnp.where(counts > 0, (offsets + counts - 1) // TM, ts)
    nt = jnp.where(counts > 0, te - ts + 1, 0)
    starts = jnp.concatenate([jnp.zeros(1, _i32), jnp.cumsum(nt)])
    total = starts[E]
    gi = jnp.arange(gmax, dtype=_i32)
    e_of = jnp.clip(jnp.searchsorted(starts[:E], gi, side="right") - 1, 0, E - 1)
    e_of = e_of.astype(_i32)
    t_of = ts[e_of] + (gi - starts[e_of])
    valid = gi < total
    t_of = jnp.clip(jnp.where(valid, t_of, t_tiles - 1), 0, t_tiles - 1)
    row_s = jnp.clip(offsets[e_of] - t_of * TM, 0, TM)
    row_e = jnp.clip(offsets[e_of] + counts[e_of] - t_of * TM, 0, TM)
    row_s = jnp.where(valid, row_s, 0)
    row_e = jnp.where(valid, row_e, 0)
    return jnp.stack([t_of, e_of, row_s, row_e]).astype(_i32)


def _pos_body(e0_ref, e1_ref, r0_ref, r1_ref, off_ref, p0_ref, p1_ref):
    iot = lax.broadcasted_iota(_i32, (RTM, E), 1)
    offb = jnp.broadcast_to(off_ref[...], (RTM, E))
    o0 = jnp.sum(jnp.where(iot == e0_ref[...], offb, 0), axis=1, keepdims=True)
    o1 = jnp.sum(jnp.where(iot == e1_ref[...], offb, 0), axis=1, keepdims=True)
    p0_ref[...] = o0 + r0_ref[...]
    p1_ref[...] = o1 + r1_ref[...]


def _positions(e0, e1, r0, r1, offsets):
    n = e0.shape[0]
    tok_spec = pl.BlockSpec((RTM, 1), lambda i: (i, 0))
    return pl.pallas_call(
        _pos_body,
        grid=(n // RTM,),
        in_specs=[tok_spec, tok_spec, tok_spec, tok_spec,
                  pl.BlockSpec((1, E), lambda i: (0, 0))],
        out_specs=(tok_spec, tok_spec),
        out_shape=(jax.ShapeDtypeStruct((n, 1), _i32),
                   jax.ShapeDtypeStruct((n, 1), _i32)),
    )(e0, e1, r0, r1, offsets)


_DISP_C = 64


def _dispatch_body(x_hbm, p0_hbm, p1_hbm, w0_hbm, w1_hbm,
                   xs_hbm, ws_hbm,
                   xv, p0v, p1v, w0v, w1v, sem):
    wid = lax.axis_index("s") * NC + lax.axis_index("c")
    n_tok = x_hbm.shape[0]
    per_w = n_tok // NW
    base = wid * per_w
    for c in range(per_w // _DISP_C):
        b = base + c * _DISP_C
        pltpu.sync_copy(x_hbm.at[pl.ds(b, _DISP_C)], xv)
        pltpu.sync_copy(p0_hbm.at[pl.ds(b, _DISP_C)], p0v)
        pltpu.sync_copy(p1_hbm.at[pl.ds(b, _DISP_C)], p1v)
        pltpu.sync_copy(w0_hbm.at[pl.ds(b, _DISP_C)], w0v)
        pltpu.sync_copy(w1_hbm.at[pl.ds(b, _DISP_C)], w1v)
        cps = [pltpu.async_copy(xv, xs_hbm.at[p0v], sem),
               pltpu.async_copy(xv, xs_hbm.at[p1v], sem),
               pltpu.async_copy(w0v, ws_hbm.at[p0v], sem),
               pltpu.async_copy(w1v, ws_hbm.at[p1v], sem)]
        for cp in cps:
            cp.wait()


def _dispatch(x_flat, p0, p1, w0, w1):
    n = x_flat.shape[0]
    m = n * TOPK
    mesh = plsc.VectorSubcoreMesh(core_axis_name="c", subcore_axis_name="s",
                                  num_cores=NC, num_subcores=NS)
    f = pl.kernel(
        _dispatch_body,
        out_type=(
            jax.ShapeDtypeStruct((m, D), _f32),
            jax.ShapeDtypeStruct((m,), _f32),
        ),
        mesh=mesh,
        scratch_types=[
            pltpu.VMEM((_DISP_C, D), _f32),
            pltpu.VMEM((_DISP_C,), _i32), pltpu.VMEM((_DISP_C,), _i32),
            pltpu.VMEM((_DISP_C,), _f32), pltpu.VMEM((_DISP_C,), _f32),
            pltpu.SemaphoreType.DMA,
        ],
    )
    return f(x_flat, p0, p1, w0, w1)


def _ffn_body(m_ref, xs_ref, ws_ref, eg_ref, eu_ref, ed_ref, y_ref):
    gidx = pl.program_id(0)
    rs = m_ref[2, gidx]
    re = m_ref[3, gidx]
    xb = xs_ref[...]
    g = _dot_nt(xb, eg_ref[0])
    u = _dot_nt(xb, eu_ref[0])
    yb = _dot_nt(_silu(g) * u, ed_ref[0]) * ws_ref[...]
    ridx = lax.broadcasted_iota(_i32, (TM, D), 0)
    mask = (ridx >= rs) & (ridx < re)
    y_ref[...] = jnp.where(mask, yb, y_ref[...])


def _ffn_grouped(meta, xs, ws, eg, eu, ed, gmax):
    m = xs.shape[0]
    grid_spec = pltpu.PrefetchScalarGridSpec(
        num_scalar_prefetch=1,
        grid=(gmax,),
        in_specs=[
            pl.BlockSpec((TM, D), lambda g, mr: (mr[0, g], 0)),
            pl.BlockSpec((TM, 1), lambda g, mr: (mr[0, g], 0)),
            pl.BlockSpec((1, DFF, D), lambda g, mr: (mr[1, g], 0, 0)),
            pl.BlockSpec((1, DFF, D), lambda g, mr: (mr[1, g], 0, 0)),
            pl.BlockSpec((1, D, DFF), lambda g, mr: (mr[1, g], 0, 0)),
        ],
        out_specs=pl.BlockSpec((TM, D), lambda g, mr: (mr[0, g], 0)),
    )
    return pl.pallas_call(
        _ffn_body,
        grid_spec=grid_spec,
        out_shape=jax.ShapeDtypeStruct((m, D), _f32),
    )(meta, xs, ws, eg, eu, ed)


_COMB_C = 32


def _combine_body(ysh_hbm, y_hbm, p0_hbm, p1_hbm,
                  out_hbm, sv, y0v, y1v, i0v, i1v, sem):
    wid = lax.axis_index("s") * NC + lax.axis_index("c")
    n_tok = ysh_hbm.shape[0]
    per_w = n_tok // NW
    base = wid * per_w
    for c in range(per_w // _COMB_C):
        b = base + c * _COMB_C
        pltpu.sync_copy(p0_hbm.at[pl.ds(b, _COMB_C)], i0v)
        pltpu.sync_copy(p1_hbm.at[pl.ds(b, _COMB_C)], i1v)
        cg0 = pltpu.async_copy(y_hbm.at[i0v], y0v, sem)
        cg1 = pltpu.async_copy(y_hbm.at[i1v], y1v, sem)
        pltpu.sync_copy(ysh_hbm.at[pl.ds(b, _COMB_C)], sv)
        cg0.wait()
        cg1.wait()

        def tok(t, _):
            for j in range(D // 16):
                sl = pl.ds(j * 16, 16)
                sv[t, sl] = sv[t, sl] + y0v[t, sl] + y1v[t, sl]
            return 0

        lax.fori_loop(0, _COMB_C, tok, 0)
        pltpu.sync_copy(sv, out_hbm.at[pl.ds(b, _COMB_C)])


def _combine(ysh, y, p0, p1):
    n = ysh.shape[0]
    mesh = plsc.VectorSubcoreMesh(core_axis_name="c", subcore_axis_name="s",
                                  num_cores=NC, num_subcores=NS)
    f = pl.kernel(
        _combine_body,
        out_type=jax.ShapeDtypeStruct((n, D), _f32),
        mesh=mesh,
        scratch_types=[
            pltpu.VMEM((_COMB_C, D), _f32),
            pltpu.VMEM((_COMB_C, D), _f32),
            pltpu.VMEM((_COMB_C, D), _f32),
            pltpu.VMEM((_COMB_C,), _i32), pltpu.VMEM((_COMB_C,), _i32),
            pltpu.SemaphoreType.DMA,
        ],
    )
    return f(ysh, y, p0, p1)


def kernel(x, gate_w, eg, eu, ed, sg, su, sd):
    b, s, d = x.shape
    n = b * s
    m = n * TOPK
    t_tiles = m // TM
    gmax = t_tiles + E - 1
    x_flat = x.reshape(n, d)

    ysh, e0, e1, r0, r1, w0, w1, counts, offsets = _router_shared(
        x_flat, gate_w, sg[0], su[0], sd[0])

    meta = _build_meta(counts, offsets, t_tiles, gmax)

    p0, p1 = _positions(e0, e1, r0, r1, offsets)

    xs, ws = _dispatch(x_flat, p0.reshape(-1), p1.reshape(-1),
                       w0.reshape(-1), w1.reshape(-1))

    y = _ffn_grouped(meta, xs, ws.reshape(m, 1), eg, eu, ed, gmax)

    out = _combine(ysh, y, p0.reshape(-1), p1.reshape(-1))
    return out.reshape(b, s, d)


# speedup vs baseline: 7.0270x; 1.0018x over previous
import functools

import jax
import jax.numpy as jnp
from jax import lax
from jax.experimental import pallas as pl
from jax.experimental.pallas import tpu as pltpu
from jax.experimental.pallas import tpu_sc as plsc

D = 1024
DFF = 512
E = 64
TOPK = 2

NC = 2
NS = 16
NW = NC * NS

RTM = 512
TM = 256

_f32 = jnp.float32
_i32 = jnp.int32


def _silu(g):
    return g * (1.0 / (1.0 + jnp.exp(-g)))


def _dot_nt(a, b, precision=None):
    return lax.dot_general(a, b, (((1,), (1,)), ((), ())),
                           preferred_element_type=_f32, precision=precision)


def _router_shared_body(x_ref, gw_ref, sg_ref, su_ref, sd_ref,
                        ysh_ref, e0_ref, e1_ref, r0_ref, r1_ref,
                        w0_ref, w1_ref, cnt_ref, off_ref, carry_ref):
    i = pl.program_id(0)
    nsteps = pl.num_programs(0)
    xb = x_ref[...]

    xb16 = xb.astype(jnp.bfloat16)
    g = _dot_nt(xb16, sg_ref[...].astype(jnp.bfloat16))
    u = _dot_nt(xb16, su_ref[...].astype(jnp.bfloat16))
    h = (_silu(g) * u).astype(jnp.bfloat16)
    ysh_ref[...] = _dot_nt(h, sd_ref[...].astype(jnp.bfloat16))

    logits = _dot_nt(xb.astype(jnp.bfloat16),
                     gw_ref[...].astype(jnp.bfloat16))
    m = jnp.max(logits, axis=1, keepdims=True)
    p = jnp.exp(logits - m)
    p = p / jnp.sum(p, axis=1, keepdims=True)
    iot = lax.broadcasted_iota(_i32, (RTM, E), 1)
    p0 = jnp.max(p, axis=1, keepdims=True)
    i0 = jnp.min(jnp.where(p == p0, iot, E), axis=1, keepdims=True)
    pm = jnp.where(iot == i0, -1.0, p)
    p1 = jnp.max(pm, axis=1, keepdims=True)
    i1 = jnp.min(jnp.where(pm == p1, iot, E), axis=1, keepdims=True)
    s = p0 + p1 + 1e-6
    w0 = p0 / s
    w1 = p1 / s

    oh = (iot == i0).astype(_f32) + (iot == i1).astype(_f32)
    ri = lax.broadcasted_iota(_i32, (RTM, RTM), 0)
    ci = lax.broadcasted_iota(_i32, (RTM, RTM), 1)
    ltri = (ri > ci).astype(_f32)
    excl = lax.dot_general(ltri, oh, (((1,), (0,)), ((), ())),
                           preferred_element_type=_f32,
                           precision=lax.Precision.HIGHEST)

    @pl.when(i == 0)
    def _():
        carry_ref[...] = jnp.zeros_like(carry_ref)

    carry = carry_ref[0:1, 0:E]
    tot = excl + carry
    r0 = jnp.sum(jnp.where(iot == i0, tot, 0.0), axis=1, keepdims=True)
    r1 = jnp.sum(jnp.where(iot == i1, tot, 0.0), axis=1, keepdims=True)
    newc = carry + jnp.sum(oh, axis=0, keepdims=True)
    carry_ref[0:1, 0:E] = newc

    e0_ref[...] = i0
    e1_ref[...] = i1
    r0_ref[...] = r0.astype(_i32)
    r1_ref[...] = r1.astype(_i32)
    w0_ref[...] = w0
    w1_ref[...] = w1

    @pl.when(i == nsteps - 1)
    def _():
        er = lax.broadcasted_iota(_i32, (E, E), 0)
        ec = lax.broadcasted_iota(_i32, (E, E), 1)
        le = (ec < er).astype(_f32)
        offs = lax.dot_general(newc, le, (((1,), (1,)), ((), ())),
                               preferred_element_type=_f32,
                               precision=lax.Precision.HIGHEST)
        cnt_ref[...] = newc.astype(_i32)
        off_ref[...] = offs.astype(_i32)


def _router_shared(x_flat, gate_w, sg2, su2, sd2):
    n = x_flat.shape[0]
    grid = (n // RTM,)
    col = lambda shape: jax.ShapeDtypeStruct(shape, _i32)
    out_shapes = (
        jax.ShapeDtypeStruct((n, D), _f32),
        col((n, 1)), col((n, 1)),
        col((n, 1)), col((n, 1)),
        jax.ShapeDtypeStruct((n, 1), _f32),
        jax.ShapeDtypeStruct((n, 1), _f32),
        col((1, E)), col((1, E)),
    )
    tok_spec = pl.BlockSpec((RTM, 1), lambda i: (i, 0))
    full = lambda shape: pl.BlockSpec(shape, lambda i: tuple(0 for _ in shape))
    return pl.pallas_call(
        _router_shared_body,
        grid=grid,
        in_specs=[
            pl.BlockSpec((RTM, D), lambda i: (i, 0)),
            full((E, D)),
            full((DFF, D)), full((DFF, D)), full((D, DFF)),
        ],
        out_specs=(
            pl.BlockSpec((RTM, D), lambda i: (i, 0)),
            tok_spec, tok_spec, tok_spec, tok_spec, tok_spec, tok_spec,
            full((1, E)), full((1, E)),
        ),
        out_shape=out_shapes,
        scratch_shapes=[pltpu.VMEM((8, 128), _f32)],
    )(x_flat, gate_w, sg2, su2, sd2)


def _build_meta(counts, offsets, t_tiles, gmax):
    counts = counts.reshape(-1).astype(_i32)
    offsets = offsets.reshape(-1).astype(_i32)
    ts = offsets // TM
    te = jnp.where(counts > 0, (offsets + counts - 1) // TM, ts)
    nt = jnp.where(counts > 0, te - ts + 1, 0)
    starts = jnp.concatenate([jnp.zeros(1, _i32), jnp.cumsum(nt)])
    total = starts[E]
    gi = jnp.arange(gmax, dtype=_i32)
    e_of = jnp.clip(jnp.searchsorted(starts[:E], gi, side="right") - 1, 0, E - 1)
    e_of = e_of.astype(_i32)
    t_of = ts[e_of] + (gi - starts[e_of])
    valid = gi < total
    t_of = jnp.clip(jnp.where(valid, t_of, t_tiles - 1), 0, t_tiles - 1)
    row_s = jnp.clip(offsets[e_of] - t_of * TM, 0, TM)
    row_e = jnp.clip(offsets[e_of] + counts[e_of] - t_of * TM, 0, TM)
    row_s = jnp.where(valid, row_s, 0)
    row_e = jnp.where(valid, row_e, 0)
    return jnp.stack([t_of, e_of, row_s, row_e]).astype(_i32)


def _pos_body(e0_ref, e1_ref, r0_ref, r1_ref, off_ref, p0_ref, p1_ref):
    iot = lax.broadcasted_iota(_i32, (RTM, E), 1)
    offb = jnp.broadcast_to(off_ref[...], (RTM, E))
    o0 = jnp.sum(jnp.where(iot == e0_ref[...], offb, 0), axis=1, keepdims=True)
    o1 = jnp.sum(jnp.where(iot == e1_ref[...], offb, 0), axis=1, keepdims=True)
    p0_ref[...] = o0 + r0_ref[...]
    p1_ref[...] = o1 + r1_ref[...]


def _positions(e0, e1, r0, r1, offsets):
    n = e0.shape[0]
    tok_spec = pl.BlockSpec((RTM, 1), lambda i: (i, 0))
    return pl.pallas_call(
        _pos_body,
        grid=(n // RTM,),
        in_specs=[tok_spec, tok_spec, tok_spec, tok_spec,
                  pl.BlockSpec((1, E), lambda i: (0, 0))],
        out_specs=(tok_spec, tok_spec),
        out_shape=(jax.ShapeDtypeStruct((n, 1), _i32),
                   jax.ShapeDtypeStruct((n, 1), _i32)),
    )(e0, e1, r0, r1, offsets)


_DISP_C = 64


def _dispatch_body(x_hbm, p0_hbm, p1_hbm, w0_hbm, w1_hbm,
                   xs_hbm, ws_hbm,
                   xv, p0v, p1v, w0v, w1v, sem):
    wid = lax.axis_index("s") * NC + lax.axis_index("c")
    n_tok = x_hbm.shape[0]
    per_w = n_tok // NW
    base = wid * per_w
    for c in range(per_w // _DISP_C):
        b = base + c * _DISP_C
        pltpu.sync_copy(x_hbm.at[pl.ds(b, _DISP_C)], xv)
        pltpu.sync_copy(p0_hbm.at[pl.ds(b, _DISP_C)], p0v)
        pltpu.sync_copy(p1_hbm.at[pl.ds(b, _DISP_C)], p1v)
        pltpu.sync_copy(w0_hbm.at[pl.ds(b, _DISP_C)], w0v)
        pltpu.sync_copy(w1_hbm.at[pl.ds(b, _DISP_C)], w1v)
        cps = [pltpu.async_copy(xv, xs_hbm.at[p0v], sem),
               pltpu.async_copy(xv, xs_hbm.at[p1v], sem),
               pltpu.async_copy(w0v, ws_hbm.at[p0v], sem),
               pltpu.async_copy(w1v, ws_hbm.at[p1v], sem)]
        for cp in cps:
            cp.wait()


def _dispatch(x_flat, p0, p1, w0, w1):
    n = x_flat.shape[0]
    m = n * TOPK
    mesh = plsc.VectorSubcoreMesh(core_axis_name="c", subcore_axis_name="s",
                                  num_cores=NC, num_subcores=NS)
    f = pl.kernel(
        _dispatch_body,
        out_type=(
            jax.ShapeDtypeStruct((m, D), _f32),
            jax.ShapeDtypeStruct((m,), _f32),
        ),
        mesh=mesh,
        scratch_types=[
            pltpu.VMEM((_DISP_C, D), _f32),
            pltpu.VMEM((_DISP_C,), _i32), pltpu.VMEM((_DISP_C,), _i32),
            pltpu.VMEM((_DISP_C,), _f32), pltpu.VMEM((_DISP_C,), _f32),
            pltpu.SemaphoreType.DMA,
        ],
    )
    return f(x_flat, p0, p1, w0, w1)


def _ffn_body(m_ref, xs_ref, ws_ref, eg_ref, eu_ref, ed_ref, y_ref):
    gidx = pl.program_id(0)
    rs = m_ref[2, gidx]
    re = m_ref[3, gidx]
    xb = xs_ref[...].astype(jnp.bfloat16)
    g = _dot_nt(xb, eg_ref[0].astype(jnp.bfloat16))
    u = _dot_nt(xb, eu_ref[0].astype(jnp.bfloat16))
    h = (_silu(g) * u).astype(jnp.bfloat16)
    yb = _dot_nt(h, ed_ref[0].astype(jnp.bfloat16)) * ws_ref[...]
    ridx = lax.broadcasted_iota(_i32, (TM, D), 0)
    mask = (ridx >= rs) & (ridx < re)
    y_ref[...] = jnp.where(mask, yb, y_ref[...])


def _ffn_grouped(meta, xs, ws, eg, eu, ed, gmax):
    m = xs.shape[0]
    grid_spec = pltpu.PrefetchScalarGridSpec(
        num_scalar_prefetch=1,
        grid=(gmax,),
        in_specs=[
            pl.BlockSpec((TM, D), lambda g, mr: (mr[0, g], 0)),
            pl.BlockSpec((TM, 1), lambda g, mr: (mr[0, g], 0)),
            pl.BlockSpec((1, DFF, D), lambda g, mr: (mr[1, g], 0, 0)),
            pl.BlockSpec((1, DFF, D), lambda g, mr: (mr[1, g], 0, 0)),
            pl.BlockSpec((1, D, DFF), lambda g, mr: (mr[1, g], 0, 0)),
        ],
        out_specs=pl.BlockSpec((TM, D), lambda g, mr: (mr[0, g], 0)),
    )
    return pl.pallas_call(
        _ffn_body,
        grid_spec=grid_spec,
        out_shape=jax.ShapeDtypeStruct((m, D), _f32),
    )(meta, xs, ws, eg, eu, ed)


_COMB_C = 32


def _combine_body(ysh_hbm, y_hbm, p0_hbm, p1_hbm,
                  out_hbm, sv, y0v, y1v, i0v, i1v, sem):
    wid = lax.axis_index("s") * NC + lax.axis_index("c")
    n_tok = ysh_hbm.shape[0]
    per_w = n_tok // NW
    base = wid * per_w
    for c in range(per_w // _COMB_C):
        b = base + c * _COMB_C
        pltpu.sync_copy(p0_hbm.at[pl.ds(b, _COMB_C)], i0v)
        pltpu.sync_copy(p1_hbm.at[pl.ds(b, _COMB_C)], i1v)
        cg0 = pltpu.async_copy(y_hbm.at[i0v], y0v, sem)
        cg1 = pltpu.async_copy(y_hbm.at[i1v], y1v, sem)
        pltpu.sync_copy(ysh_hbm.at[pl.ds(b, _COMB_C)], sv)
        cg0.wait()
        cg1.wait()

        def tok(t, _):
            for j in range(D // 16):
                sl = pl.ds(j * 16, 16)
                sv[t, sl] = sv[t, sl] + y0v[t, sl] + y1v[t, sl]
            return 0

        lax.fori_loop(0, _COMB_C, tok, 0)
        pltpu.sync_copy(sv, out_hbm.at[pl.ds(b, _COMB_C)])


def _combine(ysh, y, p0, p1):
    n = ysh.shape[0]
    mesh = plsc.VectorSubcoreMesh(core_axis_name="c", subcore_axis_name="s",
                                  num_cores=NC, num_subcores=NS)
    f = pl.kernel(
        _combine_body,
        out_type=jax.ShapeDtypeStruct((n, D), _f32),
        mesh=mesh,
        scratch_types=[
            pltpu.VMEM((_COMB_C, D), _f32),
            pltpu.VMEM((_COMB_C, D), _f32),
            pltpu.VMEM((_COMB_C, D), _f32),
            pltpu.VMEM((_COMB_C,), _i32), pltpu.VMEM((_COMB_C,), _i32),
            pltpu.SemaphoreType.DMA,
        ],
    )
    return f(ysh, y, p0, p1)


def kernel(x, gate_w, eg, eu, ed, sg, su, sd):
    b, s, d = x.shape
    n = b * s
    m = n * TOPK
    t_tiles = m // TM
    gmax = t_tiles + E - 1
    x_flat = x.reshape(n, d)

    ysh, e0, e1, r0, r1, w0, w1, counts, offsets = _router_shared(
        x_flat, gate_w, sg[0], su[0], sd[0])

    meta = _build_meta(counts, offsets, t_tiles, gmax)

    p0, p1 = _positions(e0, e1, r0, r1, offsets)

    xs, ws = _dispatch(x_flat, p0.reshape(-1), p1.reshape(-1),
                       w0.reshape(-1), w1.reshape(-1))

    y = _ffn_grouped(meta, xs, ws.reshape(m, 1), eg, eu, ed, gmax)

    out = _combine(ysh, y, p0.reshape(-1), p1.reshape(-1))
    return out.reshape(b, s, d)
